# baseline (speedup 1.0000x reference)
import functools

import jax
import jax.numpy as jnp
from jax import lax
from jax.experimental import pallas as pl
from jax.experimental.pallas import tpu as pltpu
from jax.experimental.pallas import tpu_sc as plsc

DIM = 1024
ROWS = 1000
BATCH = 16384
NUM_CORES = 2
NUM_SUBCORES = 16
NUM_WORKERS = NUM_CORES * NUM_SUBCORES
B_PER_W = BATCH // NUM_WORKERS
STAGE = 64

SP_ROWS = 256
CHUNK = 32
NSUPER = 4


@jax.jit
def _gather(timestep, pe_matrix):
    mesh = plsc.VectorSubcoreMesh(
        core_axis_name="c", subcore_axis_name="s",
        num_cores=NUM_CORES, num_subcores=NUM_SUBCORES,
    )

    @functools.partial(
        pl.kernel,
        out_type=jax.ShapeDtypeStruct((BATCH, DIM), jnp.float32),
        mesh=mesh,
        scratch_types=[
            pltpu.VMEM((B_PER_W,), jnp.int32),
            pltpu.VMEM((CHUNK, DIM), jnp.float32),
            pltpu.VMEM((CHUNK, DIM), jnp.float32),
            pltpu.VMEM_SHARED((ROWS, DIM), jnp.float32),
            pltpu.SemaphoreType.DMA,
            pltpu.SemaphoreType.DMA,
            pltpu.SemaphoreType.DMA,
            pltpu.SemaphoreType.DMA,
            pltpu.SemaphoreType.DMA,
        ],
    )
    def body(idx_hbm, table_hbm, out_hbm, idx_v, buf0, buf1, table_sh,
             ssem, g0, g1, p0, p1):
        cid = lax.axis_index("c")
        sid = lax.axis_index("s")
        wid = sid * NUM_CORES + cid
        base = wid * B_PER_W

        @pl.when(sid < NUM_SUBCORES - 1)
        def _():
            sl = pl.ds(sid * STAGE, STAGE)
            pltpu.sync_copy(table_hbm.at[sl], table_sh.at[sl])

        @pl.when(sid == NUM_SUBCORES - 1)
        def _():
            sl = pl.ds((NUM_SUBCORES - 1) * STAGE,
                       ROWS - (NUM_SUBCORES - 1) * STAGE)
            pltpu.sync_copy(table_hbm.at[sl], table_sh.at[sl])

        pltpu.sync_copy(idx_hbm.at[pl.ds(base, B_PER_W)], idx_v)
        plsc.subcore_barrier()

        tp_base = base + SP_ROWS

        def gather(c, buf, sem):
            idx_c = idx_v.at[pl.ds(SP_ROWS + c * CHUNK, CHUNK)]
            return pltpu.async_copy(table_hbm.at[idx_c], buf, sem)

        def put(c, buf, sem):
            dst = out_hbm.at[pl.ds(tp_base + c * CHUNK, CHUNK)]
            return pltpu.async_copy(buf, dst, sem)

        def sp_group(g):
            v = idx_v[pl.ds(g * 16, 16)]
            copies = []
            for e in range(16):
                r = v[e]
                copies.append(pltpu.async_copy(
                    table_sh.at[pl.ds(r, 1)],
                    out_hbm.at[pl.ds(base + g * 16 + e, 1)],
                    ssem,
                ))
            return copies

        def super_iter(k, _):
            ga = gather(2 * k, buf0, g0)
            sp = sp_group(4 * k)
            gb = gather(2 * k + 1, buf1, g1)
            sp += sp_group(4 * k + 1)
            ga.wait()
            pa = put(2 * k, buf0, p0)
            sp += sp_group(4 * k + 2)
            gb.wait()
            pb = put(2 * k + 1, buf1, p1)
            sp += sp_group(4 * k + 3)
            for c in sp:
                c.wait()
            pa.wait()
            pb.wait()
            return 0

        lax.fori_loop(0, NSUPER, super_iter, 0)

    return body(timestep, pe_matrix)


def kernel(timestep, pe_matrix):
    return _gather(timestep.astype(jnp.int32), pe_matrix)
